# baseline (speedup 1.0000x reference)
import functools

import jax
import jax.numpy as jnp
from jax import lax
from jax.experimental import pallas as pl
from jax.experimental.pallas import tpu as pltpu
from jax.experimental.pallas import tpu_sc as plsc

N = 10000
D = 128
E = 320000
NLAYERS = 3

NC = 2
NS = 16
NW = NC * NS
CH = 128
CPT = 80
NH = 2
CPH = CPT // NH
EPT = CPT * CH
EPAD = EPT * NW
MCPH = 80
NROWS2D = EPAD // CH
DCPT = NROWS2D // NS

NPAD = 10240
RPT = NPAD // NS

_mesh = plsc.VectorSubcoreMesh(core_axis_name="c", subcore_axis_name="s")


def _sc_pass2_body(g_hbm, src_hbm, dst_hbm, z_hbm, out_hbm,
                   src_v, dst_v, rows_v, acc, sem):
    c = lax.axis_index("c")
    s = lax.axis_index("s")
    pltpu.sync_copy(z_hbm, rows_v)
    for k in range(RPT // CH):
        pltpu.sync_copy(rows_v, acc.at[pl.ds(s * RPT + k * CH, CH)])
    plsc.subcore_barrier()

    for h in range(NH):
        pltpu.sync_copy(src_hbm.at[c, s, h], src_v)
        pltpu.sync_copy(dst_hbm.at[c, s, h], dst_v)

        def body(j, t):
            pltpu.async_copy(g_hbm.at[c].at[src_v.at[j]], rows_v, sem).wait()
            pltpu.sync_copy(rows_v, acc.at[dst_v.at[j]], add=True)
            return t

        lax.fori_loop(0, MCPH, body, 0)
    plsc.subcore_barrier()
    for k in range(RPT // CH):
        r0 = s * RPT + k * CH
        pltpu.sync_copy(acc.at[pl.ds(r0, CH)], rows_v)
        pltpu.sync_copy(rows_v, out_hbm.at[c, pl.ds(r0, CH)])


_sc_pass2 = pl.kernel(
    _sc_pass2_body, mesh=_mesh,
    out_type=jax.ShapeDtypeStruct((NC, NPAD, D), jnp.float32),
    scratch_types=[
        pltpu.VMEM((MCPH, CH), jnp.int32),
        pltpu.VMEM((MCPH, CH), jnp.int32),
        pltpu.VMEM((CH, D), jnp.float32),
        pltpu.VMEM_SHARED((NPAD, D), jnp.float32),
        pltpu.SemaphoreType.DMA,
    ],
)


def _sc_pass_body(g_hbm, src_hbm, dst_hbm, z_hbm, out_hbm,
                  src_v, dst_v, rows_v, acc, sem):
    c = lax.axis_index("c")
    s = lax.axis_index("s")
    wid = c * NS + s
    pltpu.sync_copy(z_hbm, rows_v)
    for k in range(RPT // CH):
        pltpu.sync_copy(rows_v, acc.at[pl.ds(s * RPT + k * CH, CH)])
    plsc.subcore_barrier()

    for h in range(NH):
        pltpu.sync_copy(src_hbm.at[wid, h], src_v)
        pltpu.sync_copy(dst_hbm.at[wid, h], dst_v)

        def body(j, t):
            pltpu.async_copy(g_hbm.at[src_v.at[j]], rows_v, sem).wait()
            pltpu.sync_copy(rows_v, acc.at[dst_v.at[j]], add=True)
            return t

        lax.fori_loop(0, CPH, body, 0)
    plsc.subcore_barrier()
    for k in range(RPT // CH):
        r0 = s * RPT + k * CH
        pltpu.sync_copy(acc.at[pl.ds(r0, CH)], rows_v)
        pltpu.sync_copy(rows_v, out_hbm.at[c, pl.ds(r0, CH)])


_sc_pass = pl.kernel(
    _sc_pass_body, mesh=_mesh,
    out_type=jax.ShapeDtypeStruct((NC, NPAD, D), jnp.float32),
    scratch_types=[
        pltpu.VMEM((CPH, CH), jnp.int32),
        pltpu.VMEM((CPH, CH), jnp.int32),
        pltpu.VMEM((CH, D), jnp.float32),
        pltpu.VMEM_SHARED((NPAD, D), jnp.float32),
        pltpu.SemaphoreType.DMA,
    ],
)


def _sc_deg_body(uq_hbm, z1_hbm, o1_hbm, out_hbm,
                 idx_v, ones_v, zb1, acc1):
    c = lax.axis_index("c")
    s = lax.axis_index("s")
    pltpu.sync_copy(uq_hbm.at[c, s], idx_v)
    pltpu.sync_copy(o1_hbm, ones_v)
    pltpu.sync_copy(z1_hbm, zb1)
    pltpu.sync_copy(zb1, acc1.at[pl.ds(s * RPT, RPT)])
    plsc.subcore_barrier()

    def body(j, t):
        pltpu.sync_copy(ones_v, acc1.at[idx_v.at[j]], add=True)
        return t

    lax.fori_loop(0, DCPT, body, 0)
    plsc.subcore_barrier()
    pltpu.sync_copy(acc1.at[pl.ds(s * RPT, RPT)], zb1)
    pltpu.sync_copy(zb1, out_hbm.at[pl.ds(c * NPAD + s * RPT, RPT)])


_sc_deg = pl.kernel(
    _sc_deg_body, mesh=_mesh,
    out_type=jax.ShapeDtypeStruct((NC * NPAD,), jnp.float32),
    scratch_types=[
        pltpu.VMEM((DCPT, CH), jnp.int32),
        pltpu.VMEM((CH,), jnp.float32),
        pltpu.VMEM((RPT,), jnp.float32),
        pltpu.VMEM_SHARED((NPAD,), jnp.float32),
    ],
)

BR = 512


def _prep_body(du_ref, dq_ref, hq_ref, ue_ref, g_ref):
    ru = lax.rsqrt(jnp.maximum(du_ref[...], 1.0))
    rq = lax.rsqrt(jnp.maximum(dq_ref[...], 1.0))
    g_ref[0] = rq * hq_ref[...]
    g_ref[1] = ru * ue_ref[...]


_prep = pl.pallas_call(
    _prep_body,
    grid=(NPAD // BR,),
    in_specs=[pl.BlockSpec((BR, D), lambda i: (i, 0))] * 4,
    out_specs=pl.BlockSpec((NC, BR, D), lambda i: (0, i, 0)),
    out_shape=jax.ShapeDtypeStruct((NC, NPAD, D), jnp.float32),
)


def _layer2_body(p_ref, du_ref, dq_ref, wu_ref, wq_ref, b_ref, g_ref):
    ru = lax.rsqrt(jnp.maximum(du_ref[...], 1.0))
    rq = lax.rsqrt(jnp.maximum(dq_ref[...], 1.0))
    hu = jnp.dot(p_ref[0] * ru, wu_ref[...],
                 preferred_element_type=jnp.float32) + b_ref[0]
    hq = jnp.dot(p_ref[1] * rq, wq_ref[...],
                 preferred_element_type=jnp.float32) + b_ref[1]
    g_ref[1] = ru * jnp.maximum(hu, 0.0)
    g_ref[0] = rq * jnp.maximum(hq, 0.0)


_layer2 = pl.pallas_call(
    _layer2_body,
    grid=(NPAD // BR,),
    in_specs=[
        pl.BlockSpec((NC, BR, D), lambda i: (0, i, 0)),
        pl.BlockSpec((BR, D), lambda i: (i, 0)),
        pl.BlockSpec((BR, D), lambda i: (i, 0)),
        pl.BlockSpec((D, D), lambda i: (0, 0)),
        pl.BlockSpec((D, D), lambda i: (0, 0)),
        pl.BlockSpec((NC, 1, D), lambda i: (0, 0, 0)),
    ],
    out_specs=pl.BlockSpec((NC, BR, D), lambda i: (0, i, 0)),
    out_shape=jax.ShapeDtypeStruct((NC, NPAD, D), jnp.float32),
)


def _layer_last_body(p_ref, dq_ref, w_ref, b_ref, o_ref):
    rq = lax.rsqrt(jnp.maximum(dq_ref[...], 1.0))
    m = (p_ref[0] + p_ref[1]) * rq
    o_ref[...] = jnp.dot(m, w_ref[...],
                         preferred_element_type=jnp.float32) + b_ref[...]


_layer_last = pl.pallas_call(
    _layer_last_body,
    grid=(NPAD // BR,),
    in_specs=[
        pl.BlockSpec((NC, BR, D), lambda i: (0, i, 0)),
        pl.BlockSpec((BR, D), lambda i: (i, 0)),
        pl.BlockSpec((D, D), lambda i: (0, 0)),
        pl.BlockSpec((1, D), lambda i: (0, 0)),
    ],
    out_specs=pl.BlockSpec((BR, D), lambda i: (i, 0)),
    out_shape=jax.ShapeDtypeStruct((NPAD, D), jnp.float32),
)


def kernel(h_qa, A_uq, u_embed, Wu, Wq, bu, bq):
    ii = A_uq.astype(jnp.int32)
    pad = jnp.full((EPAD - E,), N, jnp.int32)
    u_pad = jnp.concatenate([ii[0], pad])
    q_pad = jnp.concatenate([ii[1], pad])
    u2m = u_pad.reshape(NS, NH, MCPH, CH)
    q2m = q_pad.reshape(NS, NH, MCPH, CH)
    srcstk = jnp.stack([q2m, u2m])
    dststk = jnp.stack([u2m, q2m])
    u2s = u_pad.reshape(NW, NH, CPH, CH)
    q2s = q_pad.reshape(NW, NH, CPH, CH)
    uq = jnp.stack([u_pad, q_pad]).reshape(NC, NS, DCPT, CH)
    z2 = jnp.zeros((CH, D), jnp.float32)
    z1 = jnp.zeros((RPT,), jnp.float32)
    o1 = jnp.ones((CH,), jnp.float32)

    deg = _sc_deg(uq, z1, o1).reshape(NC, NPAD)
    DEGu = jnp.broadcast_to(deg[0][:, None], (NPAD, D))
    DEGq = jnp.broadcast_to(deg[1][:, None], (NPAD, D))
    hq_p = jnp.pad(h_qa, ((0, NPAD - N), (0, 0)))
    ue_p = jnp.pad(u_embed, ((0, NPAD - N), (0, 0)))
    g = _prep(DEGu, DEGq, hq_p, ue_p)

    for l in range(NLAYERS - 1):
        p = _sc_pass2(g, srcstk, dststk, z2)
        b2 = jnp.stack([bu[l].reshape(1, D), bq[l].reshape(1, D)])
        g = _layer2(p, DEGu, DEGq, Wu[l], Wq[l], b2)

    lastq = _sc_pass(g[1], u2s, q2s, z2)
    hq_out = _layer_last(lastq, DEGq, Wq[NLAYERS - 1],
                         bq[NLAYERS - 1].reshape(1, D))
    return hq_out[:N]


# speedup vs baseline: 2.9508x; 2.9508x over previous
import functools

import jax
import jax.numpy as jnp
from jax import lax
from jax.experimental import pallas as pl
from jax.experimental.pallas import tpu as pltpu
from jax.experimental.pallas import tpu_sc as plsc

N = 10000
D = 128
E = 320000
NLAYERS = 3

NC = 2
NS = 16
NW = NC * NS
CH = 128
CPT = 80
NH = 2
CPH = CPT // NH
EPT = CPT * CH
EPAD = EPT * NW
NROWS2D = EPAD // CH
DCPT = NROWS2D // NS

NPAD = 10240
RPT = NPAD // NS

_mesh = plsc.VectorSubcoreMesh(core_axis_name="c", subcore_axis_name="s")


def _sc_pass_body(g_hbm, src_hbm, dst_hbm, z_hbm, out_hbm,
                  src_v, dst_v, rows_a, rows_b, acc, sema, semb):
    c = lax.axis_index("c")
    s = lax.axis_index("s")
    wid = c * NS + s
    pltpu.sync_copy(z_hbm, rows_a)
    for k in range(RPT // CH):
        pltpu.sync_copy(rows_a, acc.at[pl.ds(s * RPT + k * CH, CH)])
    plsc.subcore_barrier()

    def wait_gather(buf, sem):
        pltpu.make_async_copy(g_hbm.at[src_v.at[0]], buf, sem).wait()

    for h in range(NH):
        pltpu.sync_copy(src_hbm.at[wid, h], src_v)
        pltpu.sync_copy(dst_hbm.at[wid, h], dst_v)
        pltpu.async_copy(g_hbm.at[src_v.at[0]], rows_a, sema)
        pltpu.async_copy(g_hbm.at[src_v.at[1]], rows_b, semb)

        def body(j, t):
            a = 2 * j
            wait_gather(rows_a, sema)
            pltpu.sync_copy(rows_a, acc.at[dst_v.at[a]], add=True)
            pltpu.async_copy(g_hbm.at[src_v.at[jnp.minimum(a + 2, CPH - 1)]],
                             rows_a, sema)
            b = a + 1
            wait_gather(rows_b, semb)
            pltpu.sync_copy(rows_b, acc.at[dst_v.at[b]], add=True)
            pltpu.async_copy(g_hbm.at[src_v.at[jnp.minimum(b + 2, CPH - 1)]],
                             rows_b, semb)
            return t

        lax.fori_loop(0, CPH // 2, body, 0)
        wait_gather(rows_a, sema)
        wait_gather(rows_b, semb)
    plsc.subcore_barrier()
    for k in range(RPT // CH):
        r0 = s * RPT + k * CH
        pltpu.sync_copy(acc.at[pl.ds(r0, CH)], rows_a)
        pltpu.sync_copy(rows_a, out_hbm.at[c, pl.ds(r0, CH)])


_sc_pass = pl.kernel(
    _sc_pass_body, mesh=_mesh,
    out_type=jax.ShapeDtypeStruct((NC, NPAD, D), jnp.float32),
    scratch_types=[
        pltpu.VMEM((CPH, CH), jnp.int32),
        pltpu.VMEM((CPH, CH), jnp.int32),
        pltpu.VMEM((CH, D), jnp.float32),
        pltpu.VMEM((CH, D), jnp.float32),
        pltpu.VMEM_SHARED((NPAD, D), jnp.float32),
        pltpu.SemaphoreType.DMA,
        pltpu.SemaphoreType.DMA,
    ],
)


def _sc_deg_body(uq_hbm, z1_hbm, o1_hbm, out_hbm,
                 idx_v, ones_v, zb1, acc1):
    c = lax.axis_index("c")
    s = lax.axis_index("s")
    pltpu.sync_copy(uq_hbm.at[c, s], idx_v)
    pltpu.sync_copy(o1_hbm, ones_v)
    pltpu.sync_copy(z1_hbm, zb1)
    pltpu.sync_copy(zb1, acc1.at[pl.ds(s * RPT, RPT)])
    plsc.subcore_barrier()

    def body(j, t):
        pltpu.sync_copy(ones_v, acc1.at[idx_v.at[j]], add=True)
        return t

    lax.fori_loop(0, DCPT, body, 0)
    plsc.subcore_barrier()
    pltpu.sync_copy(acc1.at[pl.ds(s * RPT, RPT)], zb1)
    pltpu.sync_copy(zb1, out_hbm.at[pl.ds(c * NPAD + s * RPT, RPT)])


_sc_deg = pl.kernel(
    _sc_deg_body, mesh=_mesh,
    out_type=jax.ShapeDtypeStruct((NC * NPAD,), jnp.float32),
    scratch_types=[
        pltpu.VMEM((DCPT, CH), jnp.int32),
        pltpu.VMEM((CH,), jnp.float32),
        pltpu.VMEM((RPT,), jnp.float32),
        pltpu.VMEM_SHARED((NPAD,), jnp.float32),
    ],
)

BR = 512


def _prep_body(du_ref, dq_ref, hq_ref, ue_ref, g_ref):
    ru = lax.rsqrt(jnp.maximum(du_ref[...], 1.0))
    rq = lax.rsqrt(jnp.maximum(dq_ref[...], 1.0))
    g_ref[0] = rq * hq_ref[...]
    g_ref[1] = ru * ue_ref[...]


_prep = pl.pallas_call(
    _prep_body,
    grid=(NPAD // BR,),
    in_specs=[pl.BlockSpec((BR, D), lambda i: (i, 0))] * 4,
    out_specs=pl.BlockSpec((NC, BR, D), lambda i: (0, i, 0)),
    out_shape=jax.ShapeDtypeStruct((NC, NPAD, D), jnp.float32),
)


def _layer2_body(pu_ref, pq_ref, du_ref, dq_ref, wu_ref, wq_ref, b_ref, g_ref):
    ru = lax.rsqrt(jnp.maximum(du_ref[...], 1.0))
    rq = lax.rsqrt(jnp.maximum(dq_ref[...], 1.0))
    hu = jnp.dot((pu_ref[0] + pu_ref[1]) * ru, wu_ref[...],
                 preferred_element_type=jnp.float32) + b_ref[0]
    hq = jnp.dot((pq_ref[0] + pq_ref[1]) * rq, wq_ref[...],
                 preferred_element_type=jnp.float32) + b_ref[1]
    g_ref[1] = ru * jnp.maximum(hu, 0.0)
    g_ref[0] = rq * jnp.maximum(hq, 0.0)


_layer2 = pl.pallas_call(
    _layer2_body,
    grid=(NPAD // BR,),
    in_specs=[
        pl.BlockSpec((NC, BR, D), lambda i: (0, i, 0)),
        pl.BlockSpec((NC, BR, D), lambda i: (0, i, 0)),
        pl.BlockSpec((BR, D), lambda i: (i, 0)),
        pl.BlockSpec((BR, D), lambda i: (i, 0)),
        pl.BlockSpec((D, D), lambda i: (0, 0)),
        pl.BlockSpec((D, D), lambda i: (0, 0)),
        pl.BlockSpec((NC, 1, D), lambda i: (0, 0, 0)),
    ],
    out_specs=pl.BlockSpec((NC, BR, D), lambda i: (0, i, 0)),
    out_shape=jax.ShapeDtypeStruct((NC, NPAD, D), jnp.float32),
)


def _layer_last_body(p_ref, dq_ref, w_ref, b_ref, o_ref):
    rq = lax.rsqrt(jnp.maximum(dq_ref[...], 1.0))
    m = (p_ref[0] + p_ref[1]) * rq
    o_ref[...] = jnp.dot(m, w_ref[...],
                         preferred_element_type=jnp.float32) + b_ref[...]


_layer_last = pl.pallas_call(
    _layer_last_body,
    grid=(NPAD // BR,),
    in_specs=[
        pl.BlockSpec((NC, BR, D), lambda i: (0, i, 0)),
        pl.BlockSpec((BR, D), lambda i: (i, 0)),
        pl.BlockSpec((D, D), lambda i: (0, 0)),
        pl.BlockSpec((1, D), lambda i: (0, 0)),
    ],
    out_specs=pl.BlockSpec((BR, D), lambda i: (i, 0)),
    out_shape=jax.ShapeDtypeStruct((NPAD, D), jnp.float32),
)


def kernel(h_qa, A_uq, u_embed, Wu, Wq, bu, bq):
    ii = A_uq.astype(jnp.int32)
    pad = N + jnp.arange(EPAD - E, dtype=jnp.int32) % (NPAD - N)
    u_pad = jnp.concatenate([ii[0], pad])
    q_pad = jnp.concatenate([ii[1], pad])
    u2s = u_pad.reshape(NW, NH, CPH, CH)
    q2s = q_pad.reshape(NW, NH, CPH, CH)
    uq = jnp.stack([u_pad, q_pad]).reshape(NC, NS, DCPT, CH)
    z2 = jnp.zeros((CH, D), jnp.float32)
    z1 = jnp.zeros((RPT,), jnp.float32)
    o1 = jnp.ones((CH,), jnp.float32)

    deg = _sc_deg(uq, z1, o1).reshape(NC, NPAD)
    DEGu = jnp.broadcast_to(deg[0][:, None], (NPAD, D))
    DEGq = jnp.broadcast_to(deg[1][:, None], (NPAD, D))
    hq_p = jnp.pad(h_qa, ((0, NPAD - N), (0, 0)))
    ue_p = jnp.pad(u_embed, ((0, NPAD - N), (0, 0)))
    g = _prep(DEGu, DEGq, hq_p, ue_p)

    for l in range(NLAYERS - 1):
        pu = _sc_pass(g[0], q2s, u2s, z2)
        pq = _sc_pass(g[1], u2s, q2s, z2)
        b2 = jnp.stack([bu[l].reshape(1, D), bq[l].reshape(1, D)])
        g = _layer2(pu, pq, DEGu, DEGq, Wu[l], Wq[l], b2)

    lastq = _sc_pass(g[1], u2s, q2s, z2)
    hq_out = _layer_last(lastq, DEGq, Wq[NLAYERS - 1],
                         bq[NLAYERS - 1].reshape(1, D))
    return hq_out[:N]


# speedup vs baseline: 4.5520x; 1.5426x over previous
import functools

import jax
import jax.numpy as jnp
from jax import lax
from jax.experimental import pallas as pl
from jax.experimental.pallas import tpu as pltpu
from jax.experimental.pallas import tpu_sc as plsc

N = 10000
D = 128
E = 320000
NLAYERS = 3

NC = 2
NS = 16
NW = NC * NS
CH = 128
CPT = 80
NH = 2
CPH = CPT // NH
EPT = CPT * CH
EPAD = EPT * NW
NROWS2D = EPAD // CH
DCPT = NROWS2D // NS

NPAD = 10240
RPT = NPAD // NS

_mesh = plsc.VectorSubcoreMesh(core_axis_name="c", subcore_axis_name="s")


def _sc_pass_body(g_hbm, src_hbm, dst_hbm, z_hbm, out_hbm,
                  src_v, dst_v, rows_a, rows_b, acc, sema, semb):
    c = lax.axis_index("c")
    s = lax.axis_index("s")
    wid = c * NS + s

    def wait_gather(buf, sem):
        pltpu.make_async_copy(g_hbm.at[src_v.at[0]], buf, sem).wait()

    pltpu.sync_copy(src_hbm.at[wid, 0], src_v)
    pltpu.sync_copy(dst_hbm.at[wid, 0], dst_v)
    pltpu.async_copy(g_hbm.at[src_v.at[1]], rows_b, semb)
    pltpu.sync_copy(z_hbm, rows_a)
    for k in range(RPT // CH):
        pltpu.sync_copy(rows_a, acc.at[pl.ds(s * RPT + k * CH, CH)])
    pltpu.async_copy(g_hbm.at[src_v.at[0]], rows_a, sema)
    plsc.subcore_barrier()

    for h in range(NH):
        if h > 0:
            pltpu.sync_copy(src_hbm.at[wid, h], src_v)
            pltpu.sync_copy(dst_hbm.at[wid, h], dst_v)
            pltpu.async_copy(g_hbm.at[src_v.at[0]], rows_a, sema)
            pltpu.async_copy(g_hbm.at[src_v.at[1]], rows_b, semb)

        def body(j, t):
            a = 2 * j
            wait_gather(rows_a, sema)
            pltpu.sync_copy(rows_a, acc.at[dst_v.at[a]], add=True)
            pltpu.async_copy(g_hbm.at[src_v.at[jnp.minimum(a + 2, CPH - 1)]],
                             rows_a, sema)
            b = a + 1
            wait_gather(rows_b, semb)
            pltpu.sync_copy(rows_b, acc.at[dst_v.at[b]], add=True)
            pltpu.async_copy(g_hbm.at[src_v.at[jnp.minimum(b + 2, CPH - 1)]],
                             rows_b, semb)
            return t

        lax.fori_loop(0, CPH // 2, body, 0)
        wait_gather(rows_a, sema)
        wait_gather(rows_b, semb)
    plsc.subcore_barrier()
    for k in range(RPT // CH):
        r0 = s * RPT + k * CH
        pltpu.sync_copy(acc.at[pl.ds(r0, CH)], rows_a)
        pltpu.sync_copy(rows_a, out_hbm.at[c, pl.ds(r0, CH)])


_sc_pass = pl.kernel(
    _sc_pass_body, mesh=_mesh,
    out_type=jax.ShapeDtypeStruct((NC, NPAD, D), jnp.float32),
    scratch_types=[
        pltpu.VMEM((CPH, CH), jnp.int32),
        pltpu.VMEM((CPH, CH), jnp.int32),
        pltpu.VMEM((CH, D), jnp.float32),
        pltpu.VMEM((CH, D), jnp.float32),
        pltpu.VMEM_SHARED((NPAD, D), jnp.float32),
        pltpu.SemaphoreType.DMA,
        pltpu.SemaphoreType.DMA,
    ],
)


def _sc_deg_body(uq_hbm, z1_hbm, o1_hbm, out_hbm,
                 idx_v, ones_v, zb1, acc1):
    c = lax.axis_index("c")
    s = lax.axis_index("s")
    pltpu.sync_copy(uq_hbm.at[c, s], idx_v)
    pltpu.sync_copy(o1_hbm, ones_v)
    pltpu.sync_copy(z1_hbm, zb1)
    pltpu.sync_copy(zb1, acc1.at[pl.ds(s * RPT, RPT)])
    plsc.subcore_barrier()

    def body(j, t):
        pltpu.sync_copy(ones_v, acc1.at[idx_v.at[j]], add=True)
        return t

    lax.fori_loop(0, DCPT, body, 0)
    plsc.subcore_barrier()
    pltpu.sync_copy(acc1.at[pl.ds(s * RPT, RPT)], zb1)
    pltpu.sync_copy(zb1, out_hbm.at[pl.ds(c * NPAD + s * RPT, RPT)])


_sc_deg = pl.kernel(
    _sc_deg_body, mesh=_mesh,
    out_type=jax.ShapeDtypeStruct((NC * NPAD,), jnp.float32),
    scratch_types=[
        pltpu.VMEM((DCPT, CH), jnp.int32),
        pltpu.VMEM((CH,), jnp.float32),
        pltpu.VMEM((RPT,), jnp.float32),
        pltpu.VMEM_SHARED((NPAD,), jnp.float32),
    ],
)

BR = 512


def _prep_body(du_ref, dq_ref, hq_ref, ue_ref, gq_ref, gu_ref):
    ru = lax.rsqrt(jnp.maximum(du_ref[...], 1.0))
    rq = lax.rsqrt(jnp.maximum(dq_ref[...], 1.0))
    gq_ref[...] = rq * hq_ref[...]
    gu_ref[...] = ru * ue_ref[...]


_prep = pl.pallas_call(
    _prep_body,
    grid=(NPAD // BR,),
    in_specs=[pl.BlockSpec((BR, D), lambda i: (i, 0))] * 4,
    out_specs=[pl.BlockSpec((BR, D), lambda i: (i, 0))] * 2,
    out_shape=[jax.ShapeDtypeStruct((NPAD, D), jnp.float32)] * 2,
)


def _layer_mid_body(p_ref, deg_ref, w_ref, b_ref, g_ref):
    r = lax.rsqrt(jnp.maximum(deg_ref[...], 1.0))
    h = jnp.dot((p_ref[0] + p_ref[1]) * r, w_ref[...],
                preferred_element_type=jnp.float32) + b_ref[...]
    g_ref[...] = r * jnp.maximum(h, 0.0)


_layer_mid = pl.pallas_call(
    _layer_mid_body,
    grid=(NPAD // BR,),
    in_specs=[
        pl.BlockSpec((NC, BR, D), lambda i: (0, i, 0)),
        pl.BlockSpec((BR, D), lambda i: (i, 0)),
        pl.BlockSpec((D, D), lambda i: (0, 0)),
        pl.BlockSpec((1, D), lambda i: (0, 0)),
    ],
    out_specs=pl.BlockSpec((BR, D), lambda i: (i, 0)),
    out_shape=jax.ShapeDtypeStruct((NPAD, D), jnp.float32),
)


def _layer_last_body(p_ref, dq_ref, w_ref, b_ref, o_ref):
    rq = lax.rsqrt(jnp.maximum(dq_ref[...], 1.0))
    m = (p_ref[0] + p_ref[1]) * rq
    o_ref[...] = jnp.dot(m, w_ref[...],
                         preferred_element_type=jnp.float32) + b_ref[...]


_layer_last = pl.pallas_call(
    _layer_last_body,
    grid=(NPAD // BR,),
    in_specs=[
        pl.BlockSpec((NC, BR, D), lambda i: (0, i, 0)),
        pl.BlockSpec((BR, D), lambda i: (i, 0)),
        pl.BlockSpec((D, D), lambda i: (0, 0)),
        pl.BlockSpec((1, D), lambda i: (0, 0)),
    ],
    out_specs=pl.BlockSpec((BR, D), lambda i: (i, 0)),
    out_shape=jax.ShapeDtypeStruct((NPAD, D), jnp.float32),
)


def kernel(h_qa, A_uq, u_embed, Wu, Wq, bu, bq):
    ii = A_uq.astype(jnp.int32)
    pad = N + jnp.arange(EPAD - E, dtype=jnp.int32) % (NPAD - N)
    u_pad = jnp.concatenate([ii[0], pad])
    q_pad = jnp.concatenate([ii[1], pad])
    u2s = u_pad.reshape(NW, NH, CPH, CH)
    q2s = q_pad.reshape(NW, NH, CPH, CH)
    uq = jnp.stack([u_pad, q_pad]).reshape(NC, NS, DCPT, CH)
    z2 = jnp.zeros((CH, D), jnp.float32)
    z1 = jnp.zeros((RPT,), jnp.float32)
    o1 = jnp.ones((CH,), jnp.float32)

    deg = _sc_deg(uq, z1, o1).reshape(NC, NPAD)
    DEGu = jnp.broadcast_to(deg[0][:, None], (NPAD, D))
    DEGq = jnp.broadcast_to(deg[1][:, None], (NPAD, D))
    hq_p = jnp.pad(h_qa, ((0, NPAD - N), (0, 0)))
    ue_p = jnp.pad(u_embed, ((0, NPAD - N), (0, 0)))
    gq, gu = _prep(DEGu, DEGq, hq_p, ue_p)

    for l in range(NLAYERS - 1):
        pu = _sc_pass(gq, q2s, u2s, z2)
        pq = _sc_pass(gu, u2s, q2s, z2)
        gu_new = _layer_mid(pu, DEGu, Wu[l], bu[l].reshape(1, D))
        gq = _layer_mid(pq, DEGq, Wq[l], bq[l].reshape(1, D))
        gu = gu_new

    lastq = _sc_pass(gu, u2s, q2s, z2)
    hq_out = _layer_last(lastq, DEGq, Wq[NLAYERS - 1],
                         bq[NLAYERS - 1].reshape(1, D))
    return hq_out[:N]
